# baseline (speedup 1.0000x reference)
import jax
import jax.numpy as jnp
from jax.experimental import pallas as pl
from jax.experimental.pallas import tpu as pltpu


def _corruption_constants(b, h):
    with jax.ensure_compile_time_eval():
        key = jax.random.key(42)
        k1, k2, k3, k4 = jax.random.split(key, 4)
        sampled = jax.random.bernoulli(k1, 0.3, (b,))
        rand_row = jax.random.randint(k2, (), 0, h)
        coin = jax.random.bernoulli(k3, 0.5)
        alpha, x_min = 2.0230031, 1.0568325
        r = jax.random.uniform(k4)
        rel = x_min * (1.0 - r) ** (-1.0 / (alpha - 1.0))
        sampled_list = [i for i in range(b) if bool(sampled[i])]
        return sampled_list, int(rand_row), bool(coin), float(rel)


def kernel(forward_input):
    b, c, h, w = forward_input.shape
    sampled, rand_row, coin, rel = _corruption_constants(b, h)

    x2 = forward_input.reshape(b * c, h * w)
    rows, lanes = x2.shape

    blk_r = 32
    assert c % blk_r == 0
    chunks_per_batch = c // blk_r
    n_chunks = rows // blk_r

    def body(in_ref, out_ref):
        scale = jnp.asarray(rel, dtype=forward_input.dtype)
        pid = pl.program_id(0)
        x = in_ref[...]
        if sampled:
            cond = False
            for bi in sampled:
                lo, hi = bi * chunks_per_batch, (bi + 1) * chunks_per_batch
                cond_i = (pid >= lo) & (pid < hi)
                cond = cond_i if cond is False else (cond | cond_i)

            @pl.when(cond)
            def _():
                if coin:
                    lane = jax.lax.broadcasted_iota(jnp.int32, x.shape, 1)
                    m = (lane % w) == rand_row
                    out_ref[...] = jnp.where(m, x * scale, x)
                else:
                    out_ref[...] = x
                    lo_l = rand_row * w
                    out_ref[:, lo_l:lo_l + w] = x[:, lo_l:lo_l + w] * scale

            @pl.when(jnp.logical_not(cond))
            def _():
                out_ref[...] = x
        else:
            out_ref[...] = x

    out2 = pl.pallas_call(
        body,
        grid=(n_chunks,),
        in_specs=[pl.BlockSpec((blk_r, lanes), lambda i: (i, 0))],
        out_specs=pl.BlockSpec((blk_r, lanes), lambda i: (i, 0)),
        out_shape=jax.ShapeDtypeStruct((rows, lanes), forward_input.dtype),
        compiler_params=pltpu.CompilerParams(
            dimension_semantics=("arbitrary",),
        ),
    )(x2)
    return out2.reshape(b, c, h, w)


# speedup vs baseline: 1.0017x; 1.0017x over previous
import jax
import jax.numpy as jnp
from jax.experimental import pallas as pl
from jax.experimental.pallas import tpu as pltpu


def _corruption_constants(b, h):
    with jax.ensure_compile_time_eval():
        key = jax.random.key(42)
        k1, k2, k3, k4 = jax.random.split(key, 4)
        sampled = jax.random.bernoulli(k1, 0.3, (b,))
        rand_row = jax.random.randint(k2, (), 0, h)
        coin = jax.random.bernoulli(k3, 0.5)
        alpha, x_min = 2.0230031, 1.0568325
        r = jax.random.uniform(k4)
        rel = x_min * (1.0 - r) ** (-1.0 / (alpha - 1.0))
        sampled_list = [i for i in range(b) if bool(sampled[i])]
        return sampled_list, int(rand_row), bool(coin), float(rel)


def kernel(forward_input):
    b, c, h, w = forward_input.shape
    sampled, rand_row, coin, rel = _corruption_constants(b, h)

    x2 = forward_input.reshape(b * c, h * w)
    rows, lanes = x2.shape

    blk_r = 32
    assert c % blk_r == 0
    chunks_per_batch = c // blk_r
    n_chunks = rows // blk_r

    def body(in_ref, out_ref):
        scale = jnp.asarray(rel, dtype=forward_input.dtype)
        pid = pl.program_id(0)
        x = in_ref[...]
        if sampled:
            cond = False
            for bi in sampled:
                lo, hi = bi * chunks_per_batch, (bi + 1) * chunks_per_batch
                cond_i = (pid >= lo) & (pid < hi)
                cond = cond_i if cond is False else (cond | cond_i)

            @pl.when(cond)
            def _():
                if coin:
                    lane = jax.lax.broadcasted_iota(jnp.int32, x.shape, 1)
                    m = (lane % w) == rand_row
                    out_ref[...] = jnp.where(m, x * scale, x)
                else:
                    out_ref[...] = x
                    lo_l = rand_row * w
                    out_ref[:, lo_l:lo_l + w] = x[:, lo_l:lo_l + w] * scale

            @pl.when(jnp.logical_not(cond))
            def _():
                out_ref[...] = x
        else:
            out_ref[...] = x

    out2 = pl.pallas_call(
        body,
        grid=(n_chunks,),
        in_specs=[pl.BlockSpec((blk_r, lanes), lambda i: (i, 0))],
        out_specs=pl.BlockSpec((blk_r, lanes), lambda i: (i, 0)),
        out_shape=jax.ShapeDtypeStruct((rows, lanes), forward_input.dtype),
        compiler_params=pltpu.CompilerParams(
            dimension_semantics=("parallel",),
        ),
    )(x2)
    return out2.reshape(b, c, h, w)
